# baseline (speedup 1.0000x reference)
import functools

import jax
import jax.numpy as jnp
from jax import lax
from jax.experimental import pallas as pl
from jax.experimental.pallas import tpu as pltpu
from jax.experimental.pallas import tpu_sc as plsc

_BATCH = 16384
_FEAT = 1024
_NC = 2
_NS = 16
_NW = _NC * _NS
_L = 16
_S = 4
_SUB = _FEAT // _S
_BPW = _BATCH // _NW
_C = 8
_NCHUNK = _BPW // _C
_DF = 4
_DC = 8


def _chunk_sum(feat_v, cent_v, accs):
    for s in range(_S):

        @plsc.parallel_loop(0, _C, carry=accs)
        def body(i, accs, s=s):
            a = list(accs)
            for k in range(_SUB // _L):
                off = s * _SUB + k * _L
                f = feat_v[i, pl.ds(off, _L)]
                g = cent_v[i, pl.ds(off, _L)]
                d = f - g
                a[k % 4] = a[k % 4] + d * d
            return tuple(a)

        accs = body
    return accs


def _sc_body(feat_hbm, lab_hbm, cent_hbm, out_hbm, idx_v,
             feat_bufs, cent_bufs, acc_v, sem_f, sem_c):
    wid = lax.axis_index("s") * _NC + lax.axis_index("c")
    base = wid * _BPW
    pltpu.sync_copy(lab_hbm.at[pl.ds(base, _BPW)], idx_v)

    def issue_f(j, b):
        pltpu.async_copy(feat_hbm.at[pl.ds(base + j * _C, _C)],
                         feat_bufs[b], sem_f[b])

    def issue_c(j, b):
        pltpu.async_copy(cent_hbm.at[idx_v.at[pl.ds(j * _C, _C)]],
                         cent_bufs[b], sem_c[b])

    def wait_f(j, b):
        pltpu.make_async_copy(
            feat_hbm.at[pl.ds(base + j * _C, _C)], feat_bufs[b], sem_f[b]).wait()

    def wait_c(j, b):
        pltpu.make_async_copy(
            cent_hbm.at[idx_v.at[pl.ds(j * _C, _C)]], cent_bufs[b], sem_c[b]).wait()

    for j in range(_DC - 1):
        issue_c(j, j)
    for j in range(_DF - 1):
        issue_f(j, j)

    def step(g, accs):
        for b in range(_DC):
            j = g * _DC + b

            @pl.when(j + _DC - 1 < _NCHUNK)
            def _():
                issue_c(j + _DC - 1, (b + _DC - 1) % _DC)

            @pl.when(j + _DF - 1 < _NCHUNK)
            def _():
                issue_f(j + _DF - 1, (b + _DF - 1) % _DF)

            wait_f(j, b % _DF)
            wait_c(j, b)
            accs = _chunk_sum(feat_bufs[b % _DF], cent_bufs[b], accs)
        return accs

    zero = jnp.zeros((_L,), jnp.float32)
    accs = lax.fori_loop(0, _NCHUNK // _DC, step, (zero, zero, zero, zero))
    acc_v[...] = (accs[0] + accs[1]) + (accs[2] + accs[3])
    pltpu.sync_copy(acc_v, out_hbm.at[wid])


@functools.partial(
    pl.kernel,
    mesh=plsc.VectorSubcoreMesh(core_axis_name="c", subcore_axis_name="s"),
    out_type=jax.ShapeDtypeStruct((_NW, _L), jnp.float32),
    scratch_types=[
        pltpu.VMEM((_BPW,), jnp.int32),
        *[pltpu.VMEM((_C, _FEAT), jnp.float32) for _ in range(_DF)],
        *[pltpu.VMEM((_C, _FEAT), jnp.float32) for _ in range(_DC)],
        pltpu.VMEM((_L,), jnp.float32),
        *[pltpu.SemaphoreType.DMA for _ in range(_DF + _DC)],
    ],
)
def _center_loss_partials(feat_hbm, lab_hbm, cent_hbm, out_hbm, idx_v, *rest):
    feat_bufs = rest[:_DF]
    cent_bufs = rest[_DF:_DF + _DC]
    acc_v = rest[_DF + _DC]
    sem_f = rest[_DF + _DC + 1:_DF + _DC + 1 + _DF]
    sem_c = rest[_DF + _DC + 1 + _DF:]
    _sc_body(feat_hbm, lab_hbm, cent_hbm, out_hbm, idx_v,
             feat_bufs, cent_bufs, acc_v, sem_f, sem_c)


def kernel(features, labels, centers):
    if labels.ndim > 1:
        labels = jnp.squeeze(labels, axis=-1)
    lab = labels.astype(jnp.int32)
    partials = _center_loss_partials(features, lab, centers)
    return jnp.sum(partials) / _BATCH


# speedup vs baseline: 1.0893x; 1.0893x over previous
import functools

import jax
import jax.numpy as jnp
from jax import lax
from jax.experimental import pallas as pl
from jax.experimental.pallas import tpu as pltpu
from jax.experimental.pallas import tpu_sc as plsc

_BATCH = 16384
_FEAT = 1024
_NC = 2
_NS = 16
_NW = _NC * _NS
_L = 16
_S = 4
_SUB = _FEAT // _S
_BPW = _BATCH // _NW
_C = 8
_NCHUNK = _BPW // _C
_DEPTH = 4


def _chunk_sum(feat_v, cent_v, accs):
    for s in range(_S):

        @plsc.parallel_loop(0, _C, carry=accs)
        def body(i, accs, s=s):
            a = list(accs)
            for k in range(_SUB // _L):
                off = s * _SUB + k * _L
                f = feat_v[i, pl.ds(off, _L)]
                g = cent_v[i, pl.ds(off, _L)]
                d = f - g
                a[k % 4] = a[k % 4] + d * d
            return tuple(a)

        accs = body
    return accs


def _sc_body(feat_hbm, lab_hbm, cent_hbm, out_hbm, idx_v,
             feat_bufs, cent_bufs, acc_v, sem_f, sem_c):
    wid = lax.axis_index("s") * _NC + lax.axis_index("c")
    base = wid * _BPW
    pltpu.sync_copy(lab_hbm.at[pl.ds(base, _BPW)], idx_v)

    def issue(j, b):
        pltpu.async_copy(feat_hbm.at[pl.ds(base + j * _C, _C)],
                         feat_bufs[b], sem_f[b])
        pltpu.async_copy(cent_hbm.at[idx_v.at[pl.ds(j * _C, _C)]],
                         cent_bufs[b], sem_c[b])

    def wait(j, b):
        pltpu.make_async_copy(
            feat_hbm.at[pl.ds(base + j * _C, _C)], feat_bufs[b], sem_f[b]).wait()
        pltpu.make_async_copy(
            cent_hbm.at[idx_v.at[pl.ds(j * _C, _C)]], cent_bufs[b], sem_c[b]).wait()

    for b in range(_DEPTH - 1):
        issue(b, b)

    def step(g, accs):
        for b in range(_DEPTH):
            j = g * _DEPTH + b

            @pl.when(j + _DEPTH - 1 < _NCHUNK)
            def _():
                issue(j + _DEPTH - 1, (b + _DEPTH - 1) % _DEPTH)

            wait(j, b)
            accs = _chunk_sum(feat_bufs[b], cent_bufs[b], accs)
        return accs

    zero = jnp.zeros((_L,), jnp.float32)
    accs = lax.fori_loop(0, _NCHUNK // _DEPTH,
                         step, (zero, zero, zero, zero))
    acc_v[...] = (accs[0] + accs[1]) + (accs[2] + accs[3])
    pltpu.sync_copy(acc_v, out_hbm.at[wid])


@functools.partial(
    pl.kernel,
    mesh=plsc.VectorSubcoreMesh(core_axis_name="c", subcore_axis_name="s"),
    out_type=jax.ShapeDtypeStruct((_NW, _L), jnp.float32),
    scratch_types=[
        pltpu.VMEM((_BPW,), jnp.int32),
        *[pltpu.VMEM((_C, _FEAT), jnp.float32) for _ in range(_DEPTH)],
        *[pltpu.VMEM((_C, _FEAT), jnp.float32) for _ in range(_DEPTH)],
        pltpu.VMEM((_L,), jnp.float32),
        *[pltpu.SemaphoreType.DMA for _ in range(2 * _DEPTH)],
    ],
)
def _center_loss_partials(feat_hbm, lab_hbm, cent_hbm, out_hbm, idx_v, *rest):
    feat_bufs = rest[:_DEPTH]
    cent_bufs = rest[_DEPTH:2 * _DEPTH]
    acc_v = rest[2 * _DEPTH]
    sem_f = rest[2 * _DEPTH + 1:2 * _DEPTH + 1 + _DEPTH]
    sem_c = rest[2 * _DEPTH + 1 + _DEPTH:]
    _sc_body(feat_hbm, lab_hbm, cent_hbm, out_hbm, idx_v,
             feat_bufs, cent_bufs, acc_v, sem_f, sem_c)


def kernel(features, labels, centers):
    if labels.ndim > 1:
        labels = jnp.squeeze(labels, axis=-1)
    lab = labels.astype(jnp.int32)
    partials = _center_loss_partials(features, lab, centers)
    return jnp.sum(partials) / _BATCH


# speedup vs baseline: 1.0934x; 1.0037x over previous
import functools

import jax
import jax.numpy as jnp
from jax import lax
from jax.experimental import pallas as pl
from jax.experimental.pallas import tpu as pltpu
from jax.experimental.pallas import tpu_sc as plsc

_BATCH = 16384
_FEAT = 1024
_NC = 2
_NS = 16
_NW = _NC * _NS
_L = 16
_S = 4
_SUB = _FEAT // _S
_BPW = _BATCH // _NW
_C = 8
_NCHUNK = _BPW // _C
_DEPTH = 4


def _chunk_sum(feat_v, cent_v, accs):
    for s in range(_S):

        @plsc.parallel_loop(0, _C, unroll=2, carry=accs)
        def body(i, accs, s=s):
            a = list(accs)
            for k in range(_SUB // _L):
                off = s * _SUB + k * _L
                f = feat_v[i, pl.ds(off, _L)]
                g = cent_v[i, pl.ds(off, _L)]
                d = f - g
                a[k % 4] = a[k % 4] + d * d
            return tuple(a)

        accs = body
    return accs


def _sc_body(feat_hbm, lab_hbm, cent_hbm, out_hbm, idx_v,
             feat_bufs, cent_bufs, acc_v, sem_f, sem_c):
    wid = lax.axis_index("s") * _NC + lax.axis_index("c")
    base = wid * _BPW
    pltpu.sync_copy(lab_hbm.at[pl.ds(base, _BPW)], idx_v)

    def issue(j, b):
        pltpu.async_copy(feat_hbm.at[pl.ds(base + j * _C, _C)],
                         feat_bufs[b], sem_f[b])
        pltpu.async_copy(cent_hbm.at[idx_v.at[pl.ds(j * _C, _C)]],
                         cent_bufs[b], sem_c[b])

    def wait(j, b):
        pltpu.make_async_copy(
            feat_hbm.at[pl.ds(base + j * _C, _C)], feat_bufs[b], sem_f[b]).wait()
        pltpu.make_async_copy(
            cent_hbm.at[idx_v.at[pl.ds(j * _C, _C)]], cent_bufs[b], sem_c[b]).wait()

    for b in range(_DEPTH - 1):
        issue(b, b)

    def step(g, accs):
        for b in range(_DEPTH):
            j = g * _DEPTH + b

            @pl.when(j + _DEPTH - 1 < _NCHUNK)
            def _():
                issue(j + _DEPTH - 1, (b + _DEPTH - 1) % _DEPTH)

            wait(j, b)
            accs = _chunk_sum(feat_bufs[b], cent_bufs[b], accs)
        return accs

    zero = jnp.zeros((_L,), jnp.float32)
    accs = lax.fori_loop(0, _NCHUNK // _DEPTH,
                         step, (zero, zero, zero, zero))
    acc_v[...] = (accs[0] + accs[1]) + (accs[2] + accs[3])
    pltpu.sync_copy(acc_v, out_hbm.at[wid])


@functools.partial(
    pl.kernel,
    mesh=plsc.VectorSubcoreMesh(core_axis_name="c", subcore_axis_name="s"),
    out_type=jax.ShapeDtypeStruct((_NW, _L), jnp.float32),
    scratch_types=[
        pltpu.VMEM((_BPW,), jnp.int32),
        *[pltpu.VMEM((_C, _FEAT), jnp.float32) for _ in range(_DEPTH)],
        *[pltpu.VMEM((_C, _FEAT), jnp.float32) for _ in range(_DEPTH)],
        pltpu.VMEM((_L,), jnp.float32),
        *[pltpu.SemaphoreType.DMA for _ in range(2 * _DEPTH)],
    ],
)
def _center_loss_partials(feat_hbm, lab_hbm, cent_hbm, out_hbm, idx_v, *rest):
    feat_bufs = rest[:_DEPTH]
    cent_bufs = rest[_DEPTH:2 * _DEPTH]
    acc_v = rest[2 * _DEPTH]
    sem_f = rest[2 * _DEPTH + 1:2 * _DEPTH + 1 + _DEPTH]
    sem_c = rest[2 * _DEPTH + 1 + _DEPTH:]
    _sc_body(feat_hbm, lab_hbm, cent_hbm, out_hbm, idx_v,
             feat_bufs, cent_bufs, acc_v, sem_f, sem_c)


def kernel(features, labels, centers):
    if labels.ndim > 1:
        labels = jnp.squeeze(labels, axis=-1)
    lab = labels.astype(jnp.int32)
    partials = _center_loss_partials(features, lab, centers)
    return jnp.sum(partials) / _BATCH
